# Initial kernel scaffold; baseline (speedup 1.0000x reference)
import jax
import jax.numpy as jnp
from jax.experimental import pallas as pl


def kernel(x, edge_index, W1, att_src1, att_dst1, b1, W2, att_src2, att_dst2, b2):
    raise NotImplementedError("write your pallas kernel here")


# speedup vs baseline: 9.3568x; 9.3568x over previous
import functools

import jax
import jax.numpy as jnp
from jax import lax
from jax.experimental import pallas as pl
from jax.experimental.pallas import tpu as pltpu
from jax.experimental.pallas import tpu_sc as plsc

N_NODES = 10000
D_IN = 128
D_HID = 128
D_OUT = 64
E_EDGES = 320000

NC = 2
NS = 16
NW = NC * NS
LANES = 16

NP = 10240
RPS = NP // NS
CH = 128
GRP = 8
E_TOT = E_EDGES + N_NODES
CPW = 88
K_EDGES = CPW * CH
E_PAD = NW * K_EDGES
RB = 1024
DX1 = D_HID + 16
DX2 = D_OUT + 16
C_ONE = 0
C_AS = 1

assert E_PAD >= E_TOT


def _sc_edge_body(D, h_hbm, ad_hbm, src_hbm, dst_hbm, z_hbm, out_hbm,
                  src_v, dst_v, ad_v, rows_v, out_sh, sem):
    DX = D + 16
    c = lax.axis_index("c")
    s = lax.axis_index("s")
    w = s * NC + c

    pltpu.sync_copy(ad_hbm, ad_v)

    pltpu.sync_copy(z_hbm, out_sh.at[pl.ds(s * RPS, RPS)])
    plsc.subcore_barrier()

    col_as = jnp.full((LANES,), D + C_AS, dtype=jnp.int32)

    def group(g, carry):
        base = w * CPW + g * GRP
        pltpu.sync_copy(src_hbm.at[pl.ds(base, GRP)], src_v)
        pltpu.sync_copy(dst_hbm.at[pl.ds(base, GRP)], dst_v)

        def chunk(k, carry2):
            cp = pltpu.async_copy(h_hbm.at[src_v.at[k]], rows_v, sem)
            cp.wait()

            def sub(t, carry3):
                dv = dst_v[k, pl.ds(t * LANES, LANES)]
                ridx = lax.iota(jnp.int32, LANES) + t * LANES
                a_s = plsc.load_gather(rows_v, [ridx, col_as])
                a_d = plsc.load_gather(ad_v, [dv])
                e = a_s + a_d
                e = jnp.where(e > 0.0, e, 0.2 * e)
                ex = jnp.exp(e)
                for l in range(LANES):
                    xr = ex[l]
                    r = t * LANES + l
                    for u in range(DX // LANES):
                        rows_v[r, pl.ds(u * LANES, LANES)] = (
                            rows_v[r, pl.ds(u * LANES, LANES)] * xr)
                return carry3

            lax.fori_loop(0, CH // LANES, sub, None)

            pltpu.sync_copy(rows_v, out_sh.at[dst_v.at[k]], add=True)
            return carry2

        lax.fori_loop(0, GRP, chunk, None)
        return carry

    lax.fori_loop(0, CPW // GRP, group, None)

    plsc.subcore_barrier()
    pltpu.sync_copy(out_sh.at[pl.ds(s * RPS, RPS)],
                    out_hbm.at[c, pl.ds(s * RPS, RPS)])


@functools.lru_cache(maxsize=None)
def _sc_edge(D):
    DX = D + 16
    mesh = plsc.VectorSubcoreMesh(core_axis_name="c", subcore_axis_name="s",
                                  num_cores=NC, num_subcores=NS)
    return pl.kernel(
        functools.partial(_sc_edge_body, D),
        out_type=jax.ShapeDtypeStruct((NC, NP, DX), jnp.float32),
        mesh=mesh,
        compiler_params=pltpu.CompilerParams(needs_layout_passes=False,
                                             use_tc_tiling_on_sc=False),
        scratch_types=[
            pltpu.VMEM((GRP, CH), jnp.int32),
            pltpu.VMEM((GRP, CH), jnp.int32),
            pltpu.VMEM((NP,), jnp.float32),
            pltpu.VMEM((CH, DX), jnp.float32),
            pltpu.VMEM_SHARED((NP, DX), jnp.float32),
            pltpu.SemaphoreType.DMA,
        ],
    )


def _tc1_body(x_ref, w_ref, as_ref, ad_ref, hx_ref, d_ref):
    h = jnp.dot(x_ref[...], w_ref[...], preferred_element_type=jnp.float32)
    a_s = jnp.dot(h, as_ref[...], preferred_element_type=jnp.float32)
    d_ref[...] = jnp.dot(h, ad_ref[...], preferred_element_type=jnp.float32)
    ones = jnp.ones((RB, 1), jnp.float32)
    padz = jnp.zeros((RB, 14), jnp.float32)
    hx_ref[...] = jnp.concatenate([h, ones, a_s, padz], axis=1)


_tc1 = pl.pallas_call(
    _tc1_body,
    grid=(NP // RB,),
    in_specs=[
        pl.BlockSpec((RB, D_IN), lambda i: (i, 0)),
        pl.BlockSpec((D_IN, D_HID), lambda i: (0, 0)),
        pl.BlockSpec((D_HID, 1), lambda i: (0, 0)),
        pl.BlockSpec((D_HID, 1), lambda i: (0, 0)),
    ],
    out_specs=[
        pl.BlockSpec((RB, DX1), lambda i: (i, 0)),
        pl.BlockSpec((RB, 1), lambda i: (i, 0)),
    ],
    out_shape=[
        jax.ShapeDtypeStruct((NP, DX1), jnp.float32),
        jax.ShapeDtypeStruct((NP, 1), jnp.float32),
    ],
)


def _tc2_body(p0_ref, p1_ref, w_ref, as_ref, ad_ref, b_ref, hx_ref, d_ref):
    acc = p0_ref[...] + p1_ref[...]
    den = jnp.maximum(acc[:, D_HID + C_ONE:D_HID + C_ONE + 1], 1e-16)
    xb = acc[:, :D_HID] / den + b_ref[...]
    xb = jnp.where(xb > 0.0, xb, 0.2 * xb)
    h = jnp.dot(xb, w_ref[...], preferred_element_type=jnp.float32)
    a_s = jnp.dot(h, as_ref[...], preferred_element_type=jnp.float32)
    d_ref[...] = jnp.dot(h, ad_ref[...], preferred_element_type=jnp.float32)
    ones = jnp.ones((RB, 1), jnp.float32)
    padz = jnp.zeros((RB, 14), jnp.float32)
    hx_ref[...] = jnp.concatenate([h, ones, a_s, padz], axis=1)


_tc2 = pl.pallas_call(
    _tc2_body,
    grid=(NP // RB,),
    in_specs=[
        pl.BlockSpec((RB, DX1), lambda i: (i, 0)),
        pl.BlockSpec((RB, DX1), lambda i: (i, 0)),
        pl.BlockSpec((D_HID, D_OUT), lambda i: (0, 0)),
        pl.BlockSpec((D_OUT, 1), lambda i: (0, 0)),
        pl.BlockSpec((D_OUT, 1), lambda i: (0, 0)),
        pl.BlockSpec((1, D_HID), lambda i: (0, 0)),
    ],
    out_specs=[
        pl.BlockSpec((RB, DX2), lambda i: (i, 0)),
        pl.BlockSpec((RB, 1), lambda i: (i, 0)),
    ],
    out_shape=[
        jax.ShapeDtypeStruct((NP, DX2), jnp.float32),
        jax.ShapeDtypeStruct((NP, 1), jnp.float32),
    ],
)


def _tc3_body(q0_ref, q1_ref, b_ref, o_ref):
    acc = q0_ref[...] + q1_ref[...]
    den = jnp.maximum(acc[:, D_OUT + C_ONE:D_OUT + C_ONE + 1], 1e-16)
    o_ref[...] = acc[:, :D_OUT] / den + b_ref[...]


_tc3 = pl.pallas_call(
    _tc3_body,
    grid=(NP // RB,),
    in_specs=[
        pl.BlockSpec((RB, DX2), lambda i: (i, 0)),
        pl.BlockSpec((RB, DX2), lambda i: (i, 0)),
        pl.BlockSpec((1, D_OUT), lambda i: (0, 0)),
    ],
    out_specs=pl.BlockSpec((RB, D_OUT), lambda i: (i, 0)),
    out_shape=jax.ShapeDtypeStruct((NP, D_OUT), jnp.float32),
)


def kernel(x, edge_index, W1, att_src1, att_dst1, b1, W2, att_src2,
           att_dst2, b2):
    loop = jnp.arange(N_NODES, dtype=jnp.int32)
    pad = jnp.full((E_PAD - E_TOT,), N_NODES, dtype=jnp.int32)
    src = jnp.concatenate([edge_index[0].astype(jnp.int32), loop, pad])
    dst = jnp.concatenate([edge_index[1].astype(jnp.int32), loop, pad])
    src = src.reshape(NW * CPW, CH)
    dst = dst.reshape(NW * CPW, CH)
    x_pad = jnp.pad(x, ((0, NP - N_NODES), (0, 0)))
    z1 = jnp.zeros((RPS, DX1), jnp.float32)
    z2 = jnp.zeros((RPS, DX2), jnp.float32)

    h1, d1 = _tc1(x_pad, W1, att_src1.reshape(D_HID, 1),
                  att_dst1.reshape(D_HID, 1))
    op1 = _sc_edge(D_HID)(h1, d1.reshape(NP), src, dst, z1)
    h2, d2 = _tc2(op1[0], op1[1], W2, att_src2.reshape(D_OUT, 1),
                  att_dst2.reshape(D_OUT, 1), b1.reshape(1, D_HID))
    op2 = _sc_edge(D_OUT)(h2, d2.reshape(NP), src, dst, z2)
    out = _tc3(op2[0], op2[1], b2.reshape(1, D_OUT))
    return out[:N_NODES]


# speedup vs baseline: 28.0057x; 2.9931x over previous
import functools

import jax
import jax.numpy as jnp
from jax import lax
from jax.experimental import pallas as pl
from jax.experimental.pallas import tpu as pltpu
from jax.experimental.pallas import tpu_sc as plsc

N_NODES = 10000
D_IN = 128
D_HID = 128
D_OUT = 64
E_EDGES = 320000

NC = 2
NS = 16
NW = NC * NS
LANES = 16

NP = 10240
RPS = NP // NS
CH = 48
GRP = 8
E_TOT = E_EDGES + N_NODES
CPW = 216
NG = CPW // GRP
K_EDGES = CPW * CH
E_PAD = NW * K_EDGES
RB = 1024
DX1 = D_HID + 16
DX2 = D_OUT + 16
C_ONE = 0
C_AS = 1

assert E_PAD >= E_TOT


def _sc_edge_body(D, h_hbm, ad_hbm, src_hbm, dst_hbm, z_hbm, out_hbm,
                  src_v, dst_v, ad_v, rows_v, out_sh, isem, gsem, ssem):
    DX = D + 16
    c = lax.axis_index("c")
    s = lax.axis_index("s")
    w = s * NC + c

    pltpu.sync_copy(ad_hbm, ad_v)

    pltpu.sync_copy(z_hbm, out_sh.at[pl.ds(s * RPS, RPS)])
    plsc.subcore_barrier()

    col_as = jnp.full((LANES,), D + C_AS, dtype=jnp.int32)

    pltpu.async_copy(src_hbm.at[pl.ds(w * CPW, GRP)], src_v.at[0], isem)
    pltpu.async_copy(dst_hbm.at[pl.ds(w * CPW, GRP)], dst_v.at[0], isem)

    def scale_chunk(ib, k, rb):
        def sub(t, carry):
            dv = dst_v[ib, k, pl.ds(t * LANES, LANES)]
            ridx = lax.iota(jnp.int32, LANES) + t * LANES
            a_s = plsc.load_gather(rows_v.at[rb], [ridx, col_as])
            a_d = plsc.load_gather(ad_v, [dv])
            e = a_s + a_d
            e = jnp.where(e > 0.0, e, 0.2 * e)
            ex = jnp.exp(e)
            for l in range(LANES):
                xr = ex[l]
                r = t * LANES + l
                for u in range(DX // LANES):
                    rows_v[rb, r, pl.ds(u * LANES, LANES)] = (
                        rows_v[rb, r, pl.ds(u * LANES, LANES)] * xr)
            return carry

        lax.fori_loop(0, CH // LANES, sub, None)

    def group(g, carry):
        ib = lax.bitwise_and(g, 1)
        base = w * CPW + g * GRP

        pltpu.make_async_copy(src_hbm.at[pl.ds(0, GRP)], src_v.at[ib],
                              isem).wait()
        pltpu.make_async_copy(dst_hbm.at[pl.ds(0, GRP)], dst_v.at[ib],
                              isem).wait()

        @pl.when(g < NG - 1)
        def _stage_next():
            nib = 1 - ib
            nbase = base + GRP
            pltpu.async_copy(src_hbm.at[pl.ds(nbase, GRP)], src_v.at[nib],
                             isem)
            pltpu.async_copy(dst_hbm.at[pl.ds(nbase, GRP)], dst_v.at[nib],
                             isem)

        gd = {0: pltpu.async_copy(h_hbm.at[src_v.at[ib, 0]], rows_v.at[0],
                                  gsem)}
        sd = {}
        for k in range(GRP):
            rb = k % 3
            gd[k].wait()
            if k + 1 < GRP:
                if k >= 2:
                    sd[k - 2].wait()
                gd[k + 1] = pltpu.async_copy(
                    h_hbm.at[src_v.at[ib, k + 1]], rows_v.at[(k + 1) % 3],
                    gsem)
            scale_chunk(ib, k, rb)
            sd[k] = pltpu.async_copy(rows_v.at[rb],
                                     out_sh.at[dst_v.at[ib, k]], ssem,
                                     add=True)
        sd[GRP - 2].wait()
        sd[GRP - 1].wait()
        return carry

    lax.fori_loop(0, NG, group, None)

    plsc.subcore_barrier()
    pltpu.sync_copy(out_sh.at[pl.ds(s * RPS, RPS)],
                    out_hbm.at[c, pl.ds(s * RPS, RPS)])


@functools.lru_cache(maxsize=None)
def _sc_edge(D):
    DX = D + 16
    mesh = plsc.VectorSubcoreMesh(core_axis_name="c", subcore_axis_name="s",
                                  num_cores=NC, num_subcores=NS)
    return pl.kernel(
        functools.partial(_sc_edge_body, D),
        out_type=jax.ShapeDtypeStruct((NC, NP, DX), jnp.float32),
        mesh=mesh,
        compiler_params=pltpu.CompilerParams(needs_layout_passes=False,
                                             use_tc_tiling_on_sc=False),
        scratch_types=[
            pltpu.VMEM((2, GRP, CH), jnp.int32),
            pltpu.VMEM((2, GRP, CH), jnp.int32),
            pltpu.VMEM((NP,), jnp.float32),
            pltpu.VMEM((3, CH, DX), jnp.float32),
            pltpu.VMEM_SHARED((NP, DX), jnp.float32),
            pltpu.SemaphoreType.DMA,
            pltpu.SemaphoreType.DMA,
            pltpu.SemaphoreType.DMA,
        ],
    )


def _tc1_body(x_ref, w_ref, as_ref, ad_ref, hx_ref, d_ref):
    h = jnp.dot(x_ref[...], w_ref[...], preferred_element_type=jnp.float32)
    a_s = jnp.dot(h, as_ref[...], preferred_element_type=jnp.float32)
    d_ref[...] = jnp.dot(h, ad_ref[...], preferred_element_type=jnp.float32)
    ones = jnp.ones((RB, 1), jnp.float32)
    padz = jnp.zeros((RB, 14), jnp.float32)
    hx_ref[...] = jnp.concatenate([h, ones, a_s, padz], axis=1)


_tc1 = pl.pallas_call(
    _tc1_body,
    grid=(NP // RB,),
    in_specs=[
        pl.BlockSpec((RB, D_IN), lambda i: (i, 0)),
        pl.BlockSpec((D_IN, D_HID), lambda i: (0, 0)),
        pl.BlockSpec((D_HID, 1), lambda i: (0, 0)),
        pl.BlockSpec((D_HID, 1), lambda i: (0, 0)),
    ],
    out_specs=[
        pl.BlockSpec((RB, DX1), lambda i: (i, 0)),
        pl.BlockSpec((RB, 1), lambda i: (i, 0)),
    ],
    out_shape=[
        jax.ShapeDtypeStruct((NP, DX1), jnp.float32),
        jax.ShapeDtypeStruct((NP, 1), jnp.float32),
    ],
)


def _tc2_body(p0_ref, p1_ref, w_ref, as_ref, ad_ref, b_ref, hx_ref, d_ref):
    acc = p0_ref[...] + p1_ref[...]
    den = jnp.maximum(acc[:, D_HID + C_ONE:D_HID + C_ONE + 1], 1e-16)
    xb = acc[:, :D_HID] / den + b_ref[...]
    xb = jnp.where(xb > 0.0, xb, 0.2 * xb)
    h = jnp.dot(xb, w_ref[...], preferred_element_type=jnp.float32)
    a_s = jnp.dot(h, as_ref[...], preferred_element_type=jnp.float32)
    d_ref[...] = jnp.dot(h, ad_ref[...], preferred_element_type=jnp.float32)
    ones = jnp.ones((RB, 1), jnp.float32)
    padz = jnp.zeros((RB, 14), jnp.float32)
    hx_ref[...] = jnp.concatenate([h, ones, a_s, padz], axis=1)


_tc2 = pl.pallas_call(
    _tc2_body,
    grid=(NP // RB,),
    in_specs=[
        pl.BlockSpec((RB, DX1), lambda i: (i, 0)),
        pl.BlockSpec((RB, DX1), lambda i: (i, 0)),
        pl.BlockSpec((D_HID, D_OUT), lambda i: (0, 0)),
        pl.BlockSpec((D_OUT, 1), lambda i: (0, 0)),
        pl.BlockSpec((D_OUT, 1), lambda i: (0, 0)),
        pl.BlockSpec((1, D_HID), lambda i: (0, 0)),
    ],
    out_specs=[
        pl.BlockSpec((RB, DX2), lambda i: (i, 0)),
        pl.BlockSpec((RB, 1), lambda i: (i, 0)),
    ],
    out_shape=[
        jax.ShapeDtypeStruct((NP, DX2), jnp.float32),
        jax.ShapeDtypeStruct((NP, 1), jnp.float32),
    ],
)


def _tc3_body(q0_ref, q1_ref, b_ref, o_ref):
    acc = q0_ref[...] + q1_ref[...]
    den = jnp.maximum(acc[:, D_OUT + C_ONE:D_OUT + C_ONE + 1], 1e-16)
    o_ref[...] = acc[:, :D_OUT] / den + b_ref[...]


_tc3 = pl.pallas_call(
    _tc3_body,
    grid=(NP // RB,),
    in_specs=[
        pl.BlockSpec((RB, DX2), lambda i: (i, 0)),
        pl.BlockSpec((RB, DX2), lambda i: (i, 0)),
        pl.BlockSpec((1, D_OUT), lambda i: (0, 0)),
    ],
    out_specs=pl.BlockSpec((RB, D_OUT), lambda i: (i, 0)),
    out_shape=jax.ShapeDtypeStruct((NP, D_OUT), jnp.float32),
)


def kernel(x, edge_index, W1, att_src1, att_dst1, b1, W2, att_src2,
           att_dst2, b2):
    loop = jnp.arange(N_NODES, dtype=jnp.int32)
    pad = jnp.full((E_PAD - E_TOT,), N_NODES, dtype=jnp.int32)
    src = jnp.concatenate([edge_index[0].astype(jnp.int32), loop, pad])
    dst = jnp.concatenate([edge_index[1].astype(jnp.int32), loop, pad])
    src = src.reshape(NW * CPW, CH)
    dst = dst.reshape(NW * CPW, CH)
    x_pad = jnp.pad(x, ((0, NP - N_NODES), (0, 0)))
    z1 = jnp.zeros((RPS, DX1), jnp.float32)
    z2 = jnp.zeros((RPS, DX2), jnp.float32)

    h1, d1 = _tc1(x_pad, W1, att_src1.reshape(D_HID, 1),
                  att_dst1.reshape(D_HID, 1))
    op1 = _sc_edge(D_HID)(h1, d1.reshape(NP), src, dst, z1)
    h2, d2 = _tc2(op1[0], op1[1], W2, att_src2.reshape(D_OUT, 1),
                  att_dst2.reshape(D_OUT, 1), b1.reshape(1, D_HID))
    op2 = _sc_edge(D_OUT)(h2, d2.reshape(NP), src, dst, z2)
    out = _tc3(op2[0], op2[1], b2.reshape(1, D_OUT))
    return out[:N_NODES]


# speedup vs baseline: 28.0184x; 1.0005x over previous
import functools

import jax
import jax.numpy as jnp
from jax import lax
from jax.experimental import pallas as pl
from jax.experimental.pallas import tpu as pltpu
from jax.experimental.pallas import tpu_sc as plsc

N_NODES = 10000
D_IN = 128
D_HID = 128
D_OUT = 64
E_EDGES = 320000

NC = 2
NS = 16
NW = NC * NS
LANES = 16

NP = 10240
RPS = NP // NS
CH = 48
GRP = 8
E_TOT = E_EDGES + N_NODES
CPW = 216
NG = CPW // GRP
K_EDGES = CPW * CH
E_PAD = NW * K_EDGES
RB = 1024
DX1 = D_HID + 16
DX2 = D_OUT + 16
C_ONE = 0
C_AS = 1

assert E_PAD >= E_TOT


def _sc_edge_body(D, h_hbm, ad_hbm, src_hbm, dst_hbm, z_hbm, out_hbm,
                  src_v, dst_v, ad_v, rows0, rows1, rows2, out_sh,
                  isem, gsem0, gsem1, gsem2, ssem0, ssem1, ssem2):
    DX = D + 16
    c = lax.axis_index("c")
    s = lax.axis_index("s")
    w = s * NC + c
    rows = (rows0, rows1, rows2)
    gsem = (gsem0, gsem1, gsem2)
    ssem = (ssem0, ssem1, ssem2)

    pltpu.sync_copy(ad_hbm, ad_v)

    pltpu.sync_copy(z_hbm, out_sh.at[pl.ds(s * RPS, RPS)])
    plsc.subcore_barrier()

    col_as = jnp.full((LANES,), D + C_AS, dtype=jnp.int32)

    def drain(sem, b):
        pltpu.make_async_copy(h_hbm.at[src_v.at[0, 0]], rows[b], sem).wait()

    def scale_chunk(ib, k, b):
        def sub(t, carry):
            dv = dst_v[ib, k, pl.ds(t * LANES, LANES)]
            ridx = lax.iota(jnp.int32, LANES) + t * LANES
            a_s = plsc.load_gather(rows[b], [ridx, col_as])
            a_d = plsc.load_gather(ad_v, [dv])
            e = a_s + a_d
            e = jnp.where(e > 0.0, e, 0.2 * e)
            ex = jnp.exp(e)
            for l in range(LANES):
                xr = ex[l]
                r = t * LANES + l
                for u in range(DX // LANES):
                    rows[b][r, pl.ds(u * LANES, LANES)] = (
                        rows[b][r, pl.ds(u * LANES, LANES)] * xr)
            return carry

        lax.fori_loop(0, CH // LANES, sub, None)

    pltpu.async_copy(src_hbm.at[pl.ds(w * CPW, GRP)], src_v.at[0], isem)
    pltpu.async_copy(dst_hbm.at[pl.ds(w * CPW, GRP)], dst_v.at[0], isem)
    pltpu.make_async_copy(src_hbm.at[pl.ds(0, GRP)], src_v.at[0], isem).wait()
    pltpu.make_async_copy(dst_hbm.at[pl.ds(0, GRP)], dst_v.at[0], isem).wait()
    pltpu.async_copy(h_hbm.at[src_v.at[0, 0]], rows[0], gsem[0])

    def group(g, carry):
        ib = lax.bitwise_and(g, 1)
        base = w * CPW + g * GRP

        @pl.when(g < NG - 1)
        def _stage_next():
            nib = 1 - ib
            nbase = base + GRP
            pltpu.async_copy(src_hbm.at[pl.ds(nbase, GRP)], src_v.at[nib],
                             isem)
            pltpu.async_copy(dst_hbm.at[pl.ds(nbase, GRP)], dst_v.at[nib],
                             isem)

        for k in range(GRP):
            b = k % 3
            drain(gsem[b], b)
            if k + 1 < GRP:
                nb = (k + 1) % 3
                if k + 1 <= 2:
                    @pl.when(g > 0)
                    def _cross_drain(nb=nb):
                        drain(ssem[nb], nb)
                else:
                    drain(ssem[nb], nb)
                pltpu.async_copy(h_hbm.at[src_v.at[ib, k + 1]], rows[nb],
                                 gsem[nb])
            scale_chunk(ib, k, b)
            pltpu.async_copy(rows[b], out_sh.at[dst_v.at[ib, k]], ssem[b],
                             add=True)

        @pl.when(g < NG - 1)
        def _prime_next():
            nib = 1 - ib
            pltpu.make_async_copy(src_hbm.at[pl.ds(0, GRP)], src_v.at[nib],
                                  isem).wait()
            pltpu.make_async_copy(dst_hbm.at[pl.ds(0, GRP)], dst_v.at[nib],
                                  isem).wait()
            drain(ssem[0], 0)
            pltpu.async_copy(h_hbm.at[src_v.at[nib, 0]], rows[0], gsem[0])

        return carry

    lax.fori_loop(0, NG, group, None)

    drain(ssem[0], 0)
    drain(ssem[1], 1)
    drain(ssem[2], 2)
    plsc.subcore_barrier()
    pltpu.sync_copy(out_sh.at[pl.ds(s * RPS, RPS)],
                    out_hbm.at[c, pl.ds(s * RPS, RPS)])


@functools.lru_cache(maxsize=None)
def _sc_edge(D):
    DX = D + 16
    mesh = plsc.VectorSubcoreMesh(core_axis_name="c", subcore_axis_name="s",
                                  num_cores=NC, num_subcores=NS)
    return pl.kernel(
        functools.partial(_sc_edge_body, D),
        out_type=jax.ShapeDtypeStruct((NC, NP, DX), jnp.float32),
        mesh=mesh,
        compiler_params=pltpu.CompilerParams(needs_layout_passes=False,
                                             use_tc_tiling_on_sc=False),
        scratch_types=[
            pltpu.VMEM((2, GRP, CH), jnp.int32),
            pltpu.VMEM((2, GRP, CH), jnp.int32),
            pltpu.VMEM((NP,), jnp.float32),
            pltpu.VMEM((CH, DX), jnp.float32),
            pltpu.VMEM((CH, DX), jnp.float32),
            pltpu.VMEM((CH, DX), jnp.float32),
            pltpu.VMEM_SHARED((NP, DX), jnp.float32),
            pltpu.SemaphoreType.DMA,
            pltpu.SemaphoreType.DMA,
            pltpu.SemaphoreType.DMA,
            pltpu.SemaphoreType.DMA,
            pltpu.SemaphoreType.DMA,
            pltpu.SemaphoreType.DMA,
            pltpu.SemaphoreType.DMA,
        ],
    )


def _tc1_body(x_ref, w_ref, as_ref, ad_ref, hx_ref, d_ref):
    h = jnp.dot(x_ref[...], w_ref[...], preferred_element_type=jnp.float32)
    a_s = jnp.dot(h, as_ref[...], preferred_element_type=jnp.float32)
    d_ref[...] = jnp.dot(h, ad_ref[...], preferred_element_type=jnp.float32)
    ones = jnp.ones((RB, 1), jnp.float32)
    padz = jnp.zeros((RB, 14), jnp.float32)
    hx_ref[...] = jnp.concatenate([h, ones, a_s, padz], axis=1)


_tc1 = pl.pallas_call(
    _tc1_body,
    grid=(NP // RB,),
    in_specs=[
        pl.BlockSpec((RB, D_IN), lambda i: (i, 0)),
        pl.BlockSpec((D_IN, D_HID), lambda i: (0, 0)),
        pl.BlockSpec((D_HID, 1), lambda i: (0, 0)),
        pl.BlockSpec((D_HID, 1), lambda i: (0, 0)),
    ],
    out_specs=[
        pl.BlockSpec((RB, DX1), lambda i: (i, 0)),
        pl.BlockSpec((RB, 1), lambda i: (i, 0)),
    ],
    out_shape=[
        jax.ShapeDtypeStruct((NP, DX1), jnp.float32),
        jax.ShapeDtypeStruct((NP, 1), jnp.float32),
    ],
)


def _tc2_body(p0_ref, p1_ref, w_ref, as_ref, ad_ref, b_ref, hx_ref, d_ref):
    acc = p0_ref[...] + p1_ref[...]
    den = jnp.maximum(acc[:, D_HID + C_ONE:D_HID + C_ONE + 1], 1e-16)
    xb = acc[:, :D_HID] / den + b_ref[...]
    xb = jnp.where(xb > 0.0, xb, 0.2 * xb)
    h = jnp.dot(xb, w_ref[...], preferred_element_type=jnp.float32)
    a_s = jnp.dot(h, as_ref[...], preferred_element_type=jnp.float32)
    d_ref[...] = jnp.dot(h, ad_ref[...], preferred_element_type=jnp.float32)
    ones = jnp.ones((RB, 1), jnp.float32)
    padz = jnp.zeros((RB, 14), jnp.float32)
    hx_ref[...] = jnp.concatenate([h, ones, a_s, padz], axis=1)


_tc2 = pl.pallas_call(
    _tc2_body,
    grid=(NP // RB,),
    in_specs=[
        pl.BlockSpec((RB, DX1), lambda i: (i, 0)),
        pl.BlockSpec((RB, DX1), lambda i: (i, 0)),
        pl.BlockSpec((D_HID, D_OUT), lambda i: (0, 0)),
        pl.BlockSpec((D_OUT, 1), lambda i: (0, 0)),
        pl.BlockSpec((D_OUT, 1), lambda i: (0, 0)),
        pl.BlockSpec((1, D_HID), lambda i: (0, 0)),
    ],
    out_specs=[
        pl.BlockSpec((RB, DX2), lambda i: (i, 0)),
        pl.BlockSpec((RB, 1), lambda i: (i, 0)),
    ],
    out_shape=[
        jax.ShapeDtypeStruct((NP, DX2), jnp.float32),
        jax.ShapeDtypeStruct((NP, 1), jnp.float32),
    ],
)


def _tc3_body(q0_ref, q1_ref, b_ref, o_ref):
    acc = q0_ref[...] + q1_ref[...]
    den = jnp.maximum(acc[:, D_OUT + C_ONE:D_OUT + C_ONE + 1], 1e-16)
    o_ref[...] = acc[:, :D_OUT] / den + b_ref[...]


_tc3 = pl.pallas_call(
    _tc3_body,
    grid=(NP // RB,),
    in_specs=[
        pl.BlockSpec((RB, DX2), lambda i: (i, 0)),
        pl.BlockSpec((RB, DX2), lambda i: (i, 0)),
        pl.BlockSpec((1, D_OUT), lambda i: (0, 0)),
    ],
    out_specs=pl.BlockSpec((RB, D_OUT), lambda i: (i, 0)),
    out_shape=jax.ShapeDtypeStruct((NP, D_OUT), jnp.float32),
)


def kernel(x, edge_index, W1, att_src1, att_dst1, b1, W2, att_src2,
           att_dst2, b2):
    loop = jnp.arange(N_NODES, dtype=jnp.int32)
    pad = jnp.full((E_PAD - E_TOT,), N_NODES, dtype=jnp.int32)
    src = jnp.concatenate([edge_index[0].astype(jnp.int32), loop, pad])
    dst = jnp.concatenate([edge_index[1].astype(jnp.int32), loop, pad])
    src = src.reshape(NW * CPW, CH)
    dst = dst.reshape(NW * CPW, CH)
    x_pad = jnp.pad(x, ((0, NP - N_NODES), (0, 0)))
    z1 = jnp.zeros((RPS, DX1), jnp.float32)
    z2 = jnp.zeros((RPS, DX2), jnp.float32)

    h1, d1 = _tc1(x_pad, W1, att_src1.reshape(D_HID, 1),
                  att_dst1.reshape(D_HID, 1))
    op1 = _sc_edge(D_HID)(h1, d1.reshape(NP), src, dst, z1)
    h2, d2 = _tc2(op1[0], op1[1], W2, att_src2.reshape(D_OUT, 1),
                  att_dst2.reshape(D_OUT, 1), b1.reshape(1, D_HID))
    op2 = _sc_edge(D_OUT)(h2, d2.reshape(NP), src, dst, z2)
    out = _tc3(op2[0], op2[1], b2.reshape(1, D_OUT))
    return out[:N_NODES]
